# baseline (speedup 1.0000x reference)
import functools

import jax
import jax.numpy as jnp
from jax import lax
from jax.experimental import pallas as pl
from jax.experimental.pallas import tpu as pltpu
from jax.experimental.pallas import tpu_sc as plsc

D = 128
NC, NS = 2, 16
NW = NC * NS
N_PAD = 10240
E_PAD = 327680
EPW = E_PAD // NW
CHUNK = 128
N_CHUNKS = EPW // CHUNK
ROWS_PER_W = N_PAD // NW
STRIPE = N_PAD // NS

_mesh = plsc.VectorSubcoreMesh(core_axis_name="c", subcore_axis_name="s")


R0 = 472
R1 = 640 - R0


@functools.partial(
    pl.kernel,
    out_type=jax.ShapeDtypeStruct((N_PAD, D), jnp.float32),
    mesh=_mesh,
    scratch_types=[
        pltpu.VMEM((R0,), jnp.int32),
        pltpu.VMEM((R0, D), jnp.float32),
        pltpu.SemaphoreType.DMA,
    ],
)
def _emb_gather(table_hbm, ids_hbm, x_hbm, idx_v, rows_v, sem):
    cid = lax.axis_index("c")
    base = lax.axis_index("s") * STRIPE

    @pl.when(cid == 0)
    def _():
        pltpu.sync_copy(ids_hbm.at[pl.ds(base, R0)], idx_v)
        pltpu.async_copy(table_hbm.at[idx_v], rows_v, sem).wait()
        pltpu.sync_copy(rows_v, x_hbm.at[pl.ds(base, R0)])

    @pl.when(cid == 1)
    def _():
        iv = idx_v.at[pl.ds(0, R1)]
        rv = rows_v.at[pl.ds(0, R1)]
        pltpu.sync_copy(ids_hbm.at[pl.ds(base + R0, R1)], iv)
        pltpu.async_copy(table_hbm.at[iv], rv, sem).wait()
        pltpu.sync_copy(rv, x_hbm.at[pl.ds(base + R0, R1)])


NBUF = 2
IR = 4
TOTAL_CHUNKS = E_PAD // CHUNK
CPP = TOTAL_CHUNKS // NS
C0 = 124
C1 = CPP - C0


@functools.partial(
    pl.kernel,
    out_type=jax.ShapeDtypeStruct((NC, N_PAD, D), jnp.float32),
    mesh=_mesh,
    scratch_types=[
        pltpu.VMEM((IR, 2, CHUNK), jnp.int32),
    ] + [pltpu.VMEM((CHUNK, D), jnp.float32)] * NBUF
      + [pltpu.SemaphoreType.DMA] * (NBUF + IR)
      + [pltpu.VMEM_SHARED((N_PAD, D), jnp.float32)],
)
def _edge_scatter(x_hbm, idx_hbm, out_hbm, ring, r0, r1,
                  g0, g1, i0, i1, i2, i3, agg_sh):
    rows = (r0, r1)
    gsem = (g0, g1)
    isem = (i0, i1, i2, i3)
    cid = lax.axis_index("c")
    sid = lax.axis_index("s")
    cpc = jnp.where(cid == 0, C0, C1)
    gbase = sid * CPP + cid * C0

    def fire_idx(k, m):
        pltpu.async_copy(idx_hbm.at[gbase + k], ring.at[m], isem[m])

    def wait_idx(k, m):
        pltpu.make_async_copy(idx_hbm.at[gbase + k], ring.at[m],
                              isem[m]).wait()

    def fire_g(k, b, m):
        pltpu.async_copy(x_hbm.at[ring.at[m, 0]], rows[b], gsem[b])

    def wait_g(k, b, m):
        pltpu.make_async_copy(x_hbm.at[ring.at[m, 0]], rows[b],
                              gsem[b]).wait()

    for m in range(IR):
        fire_idx(m, m)

    def zbody(i, _):
        for j in range(D // 16):
            rows[0][i, pl.ds(j * 16, 16)] = jnp.zeros((16,), jnp.float32)
        return ()

    lax.fori_loop(0, CHUNK, zbody, ())
    for t in range(STRIPE // CHUNK):
        pltpu.sync_copy(rows[0],
                        agg_sh.at[pl.ds(sid * STRIPE + t * CHUNK, CHUNK)])
    plsc.subcore_barrier()

    for k in range(NBUF):
        wait_idx(k, k)
        fire_g(k, k, k)

    def step(k, m, b, do_idx, do_g):
        wait_g(k, b, m)
        pltpu.sync_copy(rows[b], agg_sh.at[ring.at[m, 1]], add=True)
        if do_idx:
            fire_idx(k + IR, m)
        if do_g:
            m2 = (m + NBUF) % IR
            wait_idx(k + NBUF, m2)
            fire_g(k + NBUF, b, m2)

    def obody(k0, _):
        for m in range(IR):
            step(k0 * IR + m, m, m % NBUF, True, True)
        return ()

    lax.fori_loop(0, (cpc - IR) // IR, obody, ())
    for m in range(IR):
        k = cpc - IR + m
        step(k, m, m % NBUF, False, m + NBUF < IR)

    plsc.subcore_barrier()
    pltpu.sync_copy(agg_sh.at[pl.ds(sid * STRIPE, STRIPE)],
                    out_hbm.at[cid, pl.ds(sid * STRIPE, STRIPE)])


_BR = 1024


def _mm_body(p_ref, w_ref, b_ref, o_ref):
    x = p_ref[0] + p_ref[1]
    y = jnp.dot(x, w_ref[...], preferred_element_type=jnp.float32)
    o_ref[...] = jnp.maximum(y + b_ref[...], 0.0)


def _mm_relu(partials, W, b):
    return pl.pallas_call(
        _mm_body,
        grid=(N_PAD // _BR,),
        in_specs=[
            pl.BlockSpec((NC, _BR, D), lambda i: (0, i, 0)),
            pl.BlockSpec((D, D), lambda i: (0, 0)),
            pl.BlockSpec((1, D), lambda i: (0, 0)),
        ],
        out_specs=pl.BlockSpec((_BR, D), lambda i: (i, 0)),
        out_shape=jax.ShapeDtypeStruct((N_PAD, D), jnp.float32),
    )(partials, W, b.reshape(1, D))


def kernel(cncpt_ids, edge_index, emb_table, W1, b1, W2, b2):
    n = cncpt_ids.shape[0]
    e = edge_index.shape[1]
    ids_pad = jnp.pad(cncpt_ids.astype(jnp.int32), (0, N_PAD - n))
    src_pad = jnp.pad(edge_index[0].astype(jnp.int32),
                      (0, E_PAD - e)).reshape(NW, N_CHUNKS, CHUNK)
    dst_pad = jnp.pad(edge_index[1].astype(jnp.int32), (0, E_PAD - e),
                      constant_values=n).reshape(NW, N_CHUNKS, CHUNK)
    idx_pair = jnp.stack([src_pad.reshape(-1, CHUNK),
                          dst_pad.reshape(-1, CHUNK)],
                         axis=1)

    x = _emb_gather(emb_table, ids_pad)
    p1 = _edge_scatter(x, idx_pair)
    h1 = _mm_relu(p1, W1, b1)
    p2 = _edge_scatter(h1, idx_pair)
    h2 = _mm_relu(p2, W2, b2)
    return h2[:n]


# speedup vs baseline: 1.0007x; 1.0007x over previous
import functools

import jax
import jax.numpy as jnp
from jax import lax
from jax.experimental import pallas as pl
from jax.experimental.pallas import tpu as pltpu
from jax.experimental.pallas import tpu_sc as plsc

D = 128
NC, NS = 2, 16
NW = NC * NS
N_PAD = 10240
E_PAD = 327680
EPW = E_PAD // NW
CHUNK = 128
N_CHUNKS = EPW // CHUNK
ROWS_PER_W = N_PAD // NW
STRIPE = N_PAD // NS

_mesh = plsc.VectorSubcoreMesh(core_axis_name="c", subcore_axis_name="s")


R0 = 472
R1 = 640 - R0


@functools.partial(
    pl.kernel,
    out_type=jax.ShapeDtypeStruct((N_PAD, D), jnp.float32),
    mesh=_mesh,
    scratch_types=[
        pltpu.VMEM((R0,), jnp.int32),
        pltpu.VMEM((R0, D), jnp.float32),
        pltpu.SemaphoreType.DMA,
    ],
)
def _emb_gather(table_hbm, ids_hbm, x_hbm, idx_v, rows_v, sem):
    cid = lax.axis_index("c")
    base = lax.axis_index("s") * STRIPE

    @pl.when(cid == 0)
    def _():
        pltpu.sync_copy(ids_hbm.at[pl.ds(base, R0)], idx_v)
        pltpu.async_copy(table_hbm.at[idx_v], rows_v, sem).wait()
        pltpu.sync_copy(rows_v, x_hbm.at[pl.ds(base, R0)])

    @pl.when(cid == 1)
    def _():
        iv = idx_v.at[pl.ds(0, R1)]
        rv = rows_v.at[pl.ds(0, R1)]
        pltpu.sync_copy(ids_hbm.at[pl.ds(base + R0, R1)], iv)
        pltpu.async_copy(table_hbm.at[iv], rv, sem).wait()
        pltpu.sync_copy(rv, x_hbm.at[pl.ds(base + R0, R1)])


NBUF = 2
IR = 4
TOTAL_CHUNKS = E_PAD // CHUNK
CPP = TOTAL_CHUNKS // NS
C0 = 120
C1 = CPP - C0


@functools.partial(
    pl.kernel,
    out_type=jax.ShapeDtypeStruct((NC, N_PAD, D), jnp.float32),
    mesh=_mesh,
    scratch_types=[
        pltpu.VMEM((IR, 2, CHUNK), jnp.int32),
    ] + [pltpu.VMEM((CHUNK, D), jnp.float32)] * NBUF
      + [pltpu.SemaphoreType.DMA] * (NBUF + IR)
      + [pltpu.VMEM_SHARED((N_PAD, D), jnp.float32)],
)
def _edge_scatter(x_hbm, idx_hbm, out_hbm, ring, r0, r1,
                  g0, g1, i0, i1, i2, i3, agg_sh):
    rows = (r0, r1)
    gsem = (g0, g1)
    isem = (i0, i1, i2, i3)
    cid = lax.axis_index("c")
    sid = lax.axis_index("s")
    cpc = jnp.where(cid == 0, C0, C1)
    gbase = sid * CPP + cid * C0

    def fire_idx(k, m):
        pltpu.async_copy(idx_hbm.at[gbase + k], ring.at[m], isem[m])

    def wait_idx(k, m):
        pltpu.make_async_copy(idx_hbm.at[gbase + k], ring.at[m],
                              isem[m]).wait()

    def fire_g(k, b, m):
        pltpu.async_copy(x_hbm.at[ring.at[m, 0]], rows[b], gsem[b])

    def wait_g(k, b, m):
        pltpu.make_async_copy(x_hbm.at[ring.at[m, 0]], rows[b],
                              gsem[b]).wait()

    for m in range(IR):
        fire_idx(m, m)

    def zbody(i, _):
        for j in range(D // 16):
            rows[0][i, pl.ds(j * 16, 16)] = jnp.zeros((16,), jnp.float32)
        return ()

    lax.fori_loop(0, CHUNK, zbody, ())
    for t in range(STRIPE // CHUNK):
        pltpu.sync_copy(rows[0],
                        agg_sh.at[pl.ds(sid * STRIPE + t * CHUNK, CHUNK)])
    plsc.subcore_barrier()

    for k in range(NBUF):
        wait_idx(k, k)
        fire_g(k, k, k)

    def step(k, m, b, do_idx, do_g):
        wait_g(k, b, m)
        pltpu.sync_copy(rows[b], agg_sh.at[ring.at[m, 1]], add=True)
        if do_idx:
            fire_idx(k + IR, m)
        if do_g:
            m2 = (m + NBUF) % IR
            wait_idx(k + NBUF, m2)
            fire_g(k + NBUF, b, m2)

    def obody(k0, _):
        for m in range(IR):
            step(k0 * IR + m, m, m % NBUF, True, True)
        return ()

    lax.fori_loop(0, (cpc - IR) // IR, obody, ())
    for m in range(IR):
        k = cpc - IR + m
        step(k, m, m % NBUF, False, m + NBUF < IR)

    plsc.subcore_barrier()
    pltpu.sync_copy(agg_sh.at[pl.ds(sid * STRIPE, STRIPE)],
                    out_hbm.at[cid, pl.ds(sid * STRIPE, STRIPE)])


_BR = 1024


def _mm_body(p_ref, w_ref, b_ref, o_ref):
    x = p_ref[0] + p_ref[1]
    y = jnp.dot(x, w_ref[...], preferred_element_type=jnp.float32)
    o_ref[...] = jnp.maximum(y + b_ref[...], 0.0)


def _mm_relu(partials, W, b):
    return pl.pallas_call(
        _mm_body,
        grid=(N_PAD // _BR,),
        in_specs=[
            pl.BlockSpec((NC, _BR, D), lambda i: (0, i, 0)),
            pl.BlockSpec((D, D), lambda i: (0, 0)),
            pl.BlockSpec((1, D), lambda i: (0, 0)),
        ],
        out_specs=pl.BlockSpec((_BR, D), lambda i: (i, 0)),
        out_shape=jax.ShapeDtypeStruct((N_PAD, D), jnp.float32),
    )(partials, W, b.reshape(1, D))


def kernel(cncpt_ids, edge_index, emb_table, W1, b1, W2, b2):
    n = cncpt_ids.shape[0]
    e = edge_index.shape[1]
    ids_pad = jnp.pad(cncpt_ids.astype(jnp.int32), (0, N_PAD - n))
    src_pad = jnp.pad(edge_index[0].astype(jnp.int32),
                      (0, E_PAD - e)).reshape(NW, N_CHUNKS, CHUNK)
    dst_pad = jnp.pad(edge_index[1].astype(jnp.int32), (0, E_PAD - e),
                      constant_values=n).reshape(NW, N_CHUNKS, CHUNK)
    idx_pair = jnp.stack([src_pad.reshape(-1, CHUNK),
                          dst_pad.reshape(-1, CHUNK)],
                         axis=1)

    x = _emb_gather(emb_table, ids_pad)
    p1 = _edge_scatter(x, idx_pair)
    h1 = _mm_relu(p1, W1, b1)
    p2 = _edge_scatter(h1, idx_pair)
    h2 = _mm_relu(p2, W2, b2)
    return h2[:n]
